# Initial kernel scaffold; baseline (speedup 1.0000x reference)
import jax
import jax.numpy as jnp
from jax.experimental import pallas as pl


def kernel(x, pos_emb_table):
    raise NotImplementedError("write your pallas kernel here")


# speedup vs baseline: 4.4616x; 4.4616x over previous
import jax
import jax.numpy as jnp
from jax.experimental import pallas as pl


_BLOCK_S = 256


def _body(x_ref, emb_ref, o_ref):
    o_ref[...] = x_ref[...] + emb_ref[...][:, None, :]


def kernel(x, pos_emb_table):
    S, B, D = x.shape
    grid = (S // _BLOCK_S,)
    return pl.pallas_call(
        _body,
        grid=grid,
        in_specs=[
            pl.BlockSpec((_BLOCK_S, B, D), lambda i: (i, 0, 0)),
            pl.BlockSpec((_BLOCK_S, D), lambda i: (i, 0)),
        ],
        out_specs=pl.BlockSpec((_BLOCK_S, B, D), lambda i: (i, 0, 0)),
        out_shape=jax.ShapeDtypeStruct((S, B, D), x.dtype),
    )(x, pos_emb_table[:S])
